# Initial kernel scaffold; baseline (speedup 1.0000x reference)
import jax
import jax.numpy as jnp
from jax.experimental import pallas as pl


def kernel(x, W_ih, W_hh, b_ih, b_hh):
    raise NotImplementedError("write your pallas kernel here")


# speedup vs baseline: 22.6688x; 22.6688x over previous
import functools

import jax
import jax.numpy as jnp
from jax.experimental import pallas as pl
from jax.experimental.pallas import tpu as pltpu

H = 512
B = 8
T = 512
G4 = 4 * H
TBLK = 64


def _sigmoid(x):
    return 1.0 / (1.0 + jnp.exp(-x))


def _tanh(x):
    return 2.0 / (1.0 + jnp.exp(-2.0 * x)) - 1.0


def _proj_body(xt_ref, wih_t_ref, bias_ref, a_ref):
    a_ref[...] = (
        jnp.dot(xt_ref[...], wih_t_ref[...], preferred_element_type=jnp.float32)
        + bias_ref[...]
    )


def _rec_body(a_ref, whh_t_ref, out_ref, h_ref, c_ref):
    blk = pl.program_id(0)

    @pl.when(blk == 0)
    def _():
        h_ref[...] = jnp.zeros_like(h_ref)
        c_ref[...] = jnp.zeros_like(c_ref)

    whh_t = whh_t_ref[...]

    def step(t, carry):
        h, c = carry
        a_t = a_ref[t]
        g = a_t + jnp.dot(h, whh_t, preferred_element_type=jnp.float32)
        i = _sigmoid(g[:, 0:H])
        f = _sigmoid(g[:, H : 2 * H])
        gg = _tanh(g[:, 2 * H : 3 * H])
        o = _sigmoid(g[:, 3 * H : 4 * H])
        c = f * c + i * gg
        h = o * _tanh(c)
        return h, c

    h, c = jax.lax.fori_loop(0, TBLK, step, (h_ref[...], c_ref[...]))
    h_ref[...] = h
    c_ref[...] = c
    out_ref[...] = h


def _compact(x):
    flat = x.reshape(B, -1)

    def one(f):
        idx, = jnp.nonzero(f != 0, size=f.size)
        return f[idx]

    v = jax.vmap(one)(flat)
    return v.reshape(B, T, H).transpose(1, 0, 2)


@jax.jit
def kernel(x, W_ih, W_hh, b_ih, b_hh):
    xt = _compact(x)
    wih_t = W_ih.T
    whh_t = W_hh.T
    bias = (b_ih + b_hh)[None, :]

    a = pl.pallas_call(
        _proj_body,
        grid=(T // TBLK,),
        in_specs=[
            pl.BlockSpec((TBLK * B, H), lambda i: (i, 0)),
            pl.BlockSpec((H, G4), lambda i: (0, 0)),
            pl.BlockSpec((1, G4), lambda i: (0, 0)),
        ],
        out_specs=pl.BlockSpec((TBLK * B, G4), lambda i: (i, 0)),
        out_shape=jax.ShapeDtypeStruct((T * B, G4), jnp.float32),
    )(xt.reshape(T * B, H), wih_t, bias)

    out = pl.pallas_call(
        _rec_body,
        grid=(T // TBLK,),
        in_specs=[
            pl.BlockSpec((TBLK, B, G4), lambda i: (i, 0, 0)),
            pl.BlockSpec((H, G4), lambda i: (0, 0)),
        ],
        out_specs=pl.BlockSpec((B, H), lambda i: (0, 0)),
        out_shape=jax.ShapeDtypeStruct((B, H), jnp.float32),
        scratch_shapes=[
            pltpu.VMEM((B, H), jnp.float32),
            pltpu.VMEM((B, H), jnp.float32),
        ],
    )(a.reshape(T, B, G4), whh_t)

    return out
